# Initial kernel scaffold; baseline (speedup 1.0000x reference)
import jax
import jax.numpy as jnp
from jax.experimental import pallas as pl


def kernel(x, edge_index, batch, W0, b0, W1, b1, W2, b2, Wp1, bp1, Wp2, bp2, Wv1, bv1, Wv2, bv2):
    raise NotImplementedError("write your pallas kernel here")


# speedup vs baseline: 9.3870x; 9.3870x over previous
import functools

import jax
import jax.numpy as jnp
from jax import lax
from jax.experimental import pallas as pl
from jax.experimental.pallas import tpu as pltpu
from jax.experimental.pallas import tpu_sc as plsc

N = 10000
E = 320000
G = 128
D = 128
POL = 64

NC, NS = 2, 16
NW = NC * NS
CHUNK = 128
CH_PER_W = 79
E_PAD = NW * CH_PER_W * CHUNK
PAD_N = 10112
RPT = PAD_N // NS
DUMMY = 10008

_PREC = lax.Precision.DEFAULT


@functools.lru_cache(maxsize=None)
def _sc_deg_kernel():
    mesh = plsc.VectorSubcoreMesh(
        core_axis_name="c", subcore_axis_name="s",
        num_cores=NC, num_subcores=NS)

    @functools.partial(
        pl.kernel,
        mesh=mesh,
        out_type=jax.ShapeDtypeStruct((NC, PAD_N, D), jnp.float32),
        scratch_types=[
            pltpu.VMEM_SHARED((PAD_N, D), jnp.float32),
            pltpu.VMEM((CH_PER_W, CHUNK), jnp.int32),
            pltpu.VMEM((CHUNK, D), jnp.float32),
        ],
    )
    def body(dst2d, zrows, ones2d, out, acc, dst_v, ones_v):
        c = lax.axis_index("c")
        s = lax.axis_index("s")
        wid = c * NS + s
        base = s * RPT
        pltpu.sync_copy(dst2d.at[wid], dst_v)
        pltpu.sync_copy(ones2d, ones_v)
        pltpu.sync_copy(zrows, acc.at[pl.ds(base, RPT)])
        plsc.subcore_barrier()

        def step(j, carry):
            pltpu.sync_copy(ones_v, acc.at[dst_v.at[j]], add=True)
            return carry

        lax.fori_loop(0, CH_PER_W, step, 0)
        plsc.subcore_barrier()
        pltpu.sync_copy(acc.at[pl.ds(base, RPT)], out.at[c, pl.ds(base, RPT)])

    return body


def _sc_deg(dst2d, zrows, ones2d):
    return _sc_deg_kernel()(dst2d, zrows, ones2d)


@functools.lru_cache(maxsize=None)
def _sc_edge_kernel():
    mesh = plsc.VectorSubcoreMesh(
        core_axis_name="c", subcore_axis_name="s",
        num_cores=NC, num_subcores=NS)

    @functools.partial(
        pl.kernel,
        mesh=mesh,
        out_type=jax.ShapeDtypeStruct((NC, PAD_N, D), jnp.float32),
        scratch_types=[
            pltpu.VMEM_SHARED((PAD_N, D), jnp.float32),
            pltpu.VMEM((CH_PER_W, CHUNK), jnp.int32),
            pltpu.VMEM((CH_PER_W, CHUNK), jnp.int32),
            pltpu.VMEM((CHUNK, D), jnp.float32),
        ],
    )
    def body(hp, src2d, dst2d, zrows, out, acc, src_v, dst_v, rows):
        c = lax.axis_index("c")
        s = lax.axis_index("s")
        wid = c * NS + s
        base = s * RPT
        pltpu.sync_copy(src2d.at[wid], src_v)
        pltpu.sync_copy(dst2d.at[wid], dst_v)
        pltpu.sync_copy(zrows, acc.at[pl.ds(base, RPT)])
        plsc.subcore_barrier()

        def step(j, carry):
            pltpu.sync_copy(hp.at[src_v.at[j]], rows)
            pltpu.sync_copy(rows, acc.at[dst_v.at[j]], add=True)
            return carry

        lax.fori_loop(0, CH_PER_W, step, 0)
        plsc.subcore_barrier()
        pltpu.sync_copy(acc.at[pl.ds(base, RPT)], out.at[c, pl.ds(base, RPT)])

    return body


def _sc_edge(hp, src2d, dst2d, zrows):
    return _sc_edge_kernel()(hp, src2d, dst2d, zrows)


R = 400
GRID = N // R


def _tc_a_body(x_ref, w_ref, degp_ref, o_ref, dis_ref):
    degp = degp_ref[...]
    deg = degp[0][:, 0:1] + degp[1][:, 0:1] + 1.0
    dis = lax.rsqrt(deg)
    dis_ref[...] = jnp.broadcast_to(dis, (R, 8))
    u = jnp.dot(x_ref[...], w_ref[...], precision=_PREC,
                preferred_element_type=jnp.float32)
    o_ref[...] = u * dis


def _tc_b_body(scat_ref, hp_ref, dis_ref, b_ref, w_ref, o_ref):
    dis = dis_ref[...][:, 0:1]
    t = (scat_ref[0] + scat_ref[1] + hp_ref[...]) * dis + b_ref[...]
    h = jnp.maximum(t, 0.0)
    u = jnp.dot(h, w_ref[...], precision=_PREC,
                preferred_element_type=jnp.float32)
    o_ref[...] = u * dis


def _tc_c_body(scat_ref, hp_ref, dis_ref, b_ref, batch_ref, gsum_ref, cnt_ref):
    dis = dis_ref[...][:, 0:1]
    t = (scat_ref[0] + scat_ref[1] + hp_ref[...]) * dis + b_ref[...]
    h = jnp.maximum(t, 0.0)
    bb = batch_ref[...][:, 0:1]
    gid = lax.broadcasted_iota(jnp.int32, (R, G), 1)
    m = (bb == gid).astype(jnp.float32)

    @pl.when(pl.program_id(0) == 0)
    def _():
        gsum_ref[...] = jnp.zeros_like(gsum_ref)
        cnt_ref[...] = jnp.zeros_like(cnt_ref)

    gsum_ref[...] += lax.dot_general(m, h, (((0,), (0,)), ((), ())),
                                     precision=_PREC,
                                     preferred_element_type=jnp.float32)
    cnt_ref[...] += lax.dot_general(m, jnp.ones((R, 8), jnp.float32),
                                    (((0,), (0,)), ((), ())),
                                    precision=_PREC,
                                    preferred_element_type=jnp.float32)


def _tc_d_body(gsum_ref, cnt_ref, wp1_ref, bp1_ref, wp2_ref, bp2_ref,
               wv1_ref, bv1_ref, wv2_ref, bv2_ref, pol_ref, val_ref):
    cnt = jnp.maximum(cnt_ref[...][:, 0:1], 1.0)
    g = gsum_ref[...] / cnt
    p = jnp.maximum(jnp.dot(g, wp1_ref[...], precision=_PREC,
                            preferred_element_type=jnp.float32)
                    + bp1_ref[...], 0.0)
    logits = jnp.dot(p, wp2_ref[...], precision=_PREC,
                     preferred_element_type=jnp.float32) + bp2_ref[...]
    mx = jnp.max(logits, axis=1, keepdims=True)
    ex = jnp.exp(logits - mx)
    pol_ref[...] = ex / jnp.sum(ex, axis=1, keepdims=True)
    v = jnp.maximum(jnp.dot(g, wv1_ref[...], precision=_PREC,
                            preferred_element_type=jnp.float32)
                    + bv1_ref[...], 0.0)
    val_ref[...] = jnp.tanh(jnp.dot(v, wv2_ref[...], precision=_PREC,
                                    preferred_element_type=jnp.float32)
                            + bv2_ref[...])


def _tc_a(x, w, degp):
    return pl.pallas_call(
        _tc_a_body,
        grid=(GRID,),
        in_specs=[
            pl.BlockSpec((R, D), lambda i: (i, 0)),
            pl.BlockSpec((D, D), lambda i: (0, 0)),
            pl.BlockSpec((NC, R, D), lambda i: (0, i, 0)),
        ],
        out_specs=[
            pl.BlockSpec((R, D), lambda i: (i, 0)),
            pl.BlockSpec((R, 8), lambda i: (i, 0)),
        ],
        out_shape=[
            jax.ShapeDtypeStruct((N, D), jnp.float32),
            jax.ShapeDtypeStruct((N, 8), jnp.float32),
        ],
    )(x, w, degp)


def _tc_b(scat, hp, dis8, b, w):
    return pl.pallas_call(
        _tc_b_body,
        grid=(GRID,),
        in_specs=[
            pl.BlockSpec((NC, R, D), lambda i: (0, i, 0)),
            pl.BlockSpec((R, D), lambda i: (i, 0)),
            pl.BlockSpec((R, 8), lambda i: (i, 0)),
            pl.BlockSpec((1, D), lambda i: (0, 0)),
            pl.BlockSpec((D, D), lambda i: (0, 0)),
        ],
        out_specs=pl.BlockSpec((R, D), lambda i: (i, 0)),
        out_shape=jax.ShapeDtypeStruct((N, D), jnp.float32),
    )(scat, hp, dis8, b, w)


def _tc_c(scat, hp, dis8, b, batch8):
    return pl.pallas_call(
        _tc_c_body,
        grid=(GRID,),
        in_specs=[
            pl.BlockSpec((NC, R, D), lambda i: (0, i, 0)),
            pl.BlockSpec((R, D), lambda i: (i, 0)),
            pl.BlockSpec((R, 8), lambda i: (i, 0)),
            pl.BlockSpec((1, D), lambda i: (0, 0)),
            pl.BlockSpec((R, 8), lambda i: (i, 0)),
        ],
        out_specs=[
            pl.BlockSpec((G, D), lambda i: (0, 0)),
            pl.BlockSpec((G, 8), lambda i: (0, 0)),
        ],
        out_shape=[
            jax.ShapeDtypeStruct((G, D), jnp.float32),
            jax.ShapeDtypeStruct((G, 8), jnp.float32),
        ],
    )(scat, hp, dis8, b, batch8)


def _tc_d(gsum, cnt, wp1, bp1, wp2, bp2, wv1, bv1, wv2, bv2):
    return pl.pallas_call(
        _tc_d_body,
        out_shape=[
            jax.ShapeDtypeStruct((G, POL), jnp.float32),
            jax.ShapeDtypeStruct((G, 1), jnp.float32),
        ],
    )(gsum, cnt, wp1, bp1, wp2, bp2, wv1, bv1, wv2, bv2)


def kernel(x, edge_index, batch, W0, b0, W1, b1, W2, b2,
           Wp1, bp1, Wp2, bp2, Wv1, bv1, Wv2, bv2):
    src = edge_index[0]
    dst = edge_index[1]
    pad = E_PAD - E
    src2d = jnp.concatenate(
        [src, jnp.zeros((pad,), jnp.int32)]).reshape(NW, CH_PER_W, CHUNK)
    dst2d = jnp.concatenate(
        [dst, jnp.full((pad,), DUMMY, jnp.int32)]).reshape(NW, CH_PER_W, CHUNK)

    zrows = jnp.zeros((RPT, D), jnp.float32)
    ones2d = jnp.ones((CHUNK, D), jnp.float32)
    batch8 = jnp.tile(batch[:, None], (1, 8))

    degp = _sc_deg(dst2d, zrows, ones2d)

    hp1, dis8 = _tc_a(x, W0, degp)
    scat1 = _sc_edge(hp1, src2d, dst2d, zrows)
    hp2 = _tc_b(scat1, hp1, dis8, b0.reshape(1, D), W1)
    scat2 = _sc_edge(hp2, src2d, dst2d, zrows)
    hp3 = _tc_b(scat2, hp2, dis8, b1.reshape(1, D), W2)
    scat3 = _sc_edge(hp3, src2d, dst2d, zrows)

    gsum, cnt = _tc_c(scat3, hp3, dis8, b2.reshape(1, D), batch8)
    policy, value = _tc_d(gsum, cnt,
                          Wp1, bp1.reshape(1, -1), Wp2, bp2.reshape(1, -1),
                          Wv1, bv1.reshape(1, -1), Wv2, bv2.reshape(1, 1))
    return (policy, value)
